# initial kernel scaffold (unmeasured)
import jax
import jax.numpy as jnp
from jax import lax
from jax.experimental import pallas as pl
from jax.experimental.pallas import tpu as pltpu

N_DEV = 4
SQ, DM, HQ, DH = 2048, 1024, 8, 128
CHUNK = 512
NCH = SQ // CHUNK
BAND = 768
WIN = 128
HALO = 128
KB = 2048 + HALO
HALF = CHUNK // 2
SCALE = 0.08838834764831843
F32, BF16 = jnp.float32, jnp.bfloat16
MESH = pl.DeviceIdType.MESH


def kernel(x, Wq, K_ext, V_ext, Wo):
    def body(x_hbm, wq_ref, k_ref, v_ref, wo_ref, out_ref,
             kb_ref, vb_ref, outb_ref, xst_ref, ctx_ref, hk_ref, hv_ref,
             halo_recv, halo_send, csend, crecv, fwda, fwdb, copy_sem):
        me = lax.axis_index("i")

        bsem = pltpu.get_barrier_semaphore()
        for off in (1, 2, 3):
            pl.semaphore_signal(
                bsem, inc=1, device_id=((me + off) % N_DEV,),
                device_id_type=MESH)
        pl.semaphore_wait(bsem, N_DEV - 1)

        def halo_desc(band_ref, src_ref, i):
            return pltpu.make_async_remote_copy(
                src_ref=src_ref,
                dst_ref=band_ref.at[:, pl.ds(2048, HALO), :],
                send_sem=halo_send.at[i], recv_sem=halo_recv.at[i],
                device_id=(0,), device_id_type=MESH)

        def chunk_desc(c, ti, tgt):
            sl = outb_ref.at[pl.ds(c * CHUNK, CHUNK), :]
            return pltpu.make_async_remote_copy(
                src_ref=sl, dst_ref=sl,
                send_sem=csend.at[c, ti], recv_sem=crecv.at[c],
                device_id=(tgt,), device_id_type=MESH)

        def fwd_desc(c, off, ti, sem, tgt):
            sl = outb_ref.at[pl.ds(c * CHUNK + off, HALF), :]
            return pltpu.make_async_remote_copy(
                src_ref=sl, dst_ref=sl,
                send_sem=csend.at[c, ti], recv_sem=sem.at[c],
                device_id=(tgt,), device_id_type=MESH)

        @pl.when(me == 0)
        def _dev0():
            wq_b = wq_ref[:].astype(BF16)
            wo_b = wo_ref[:].astype(BF16)
            for h in range(HQ):
                kb_ref[h, pl.ds(0, 2048), :] = k_ref[0, :, h, :].astype(BF16)
                vb_ref[h, pl.ds(0, 2048), :] = v_ref[0, :, h, :].astype(BF16)
            sends = []
            for c in range(NCH):
                cp = pltpu.make_async_copy(
                    x_hbm.at[0, pl.ds(c * CHUNK, CHUNK), :], xst_ref,
                    copy_sem)
                cp.start()
                cp.wait()
                q = jnp.dot(xst_ref[:].astype(BF16), wq_b,
                            preferred_element_type=F32)
                qb = q.astype(BF16)
                if c == NCH - 1:
                    halo_desc(kb_ref, hk_ref, 0).wait_recv()
                    halo_desc(vb_ref, hv_ref, 1).wait_recv()
                ks = max(0, c * CHUNK - WIN)
                qi = c * CHUNK + lax.broadcasted_iota(
                    jnp.int32, (CHUNK, BAND), 0)
                ki = ks + lax.broadcasted_iota(jnp.int32, (CHUNK, BAND), 1)
                mask = jnp.abs(qi - ki) <= WIN
                for h in range(HQ):
                    qh = qb[:, h * DH:(h + 1) * DH]
                    kh = kb_ref[h, pl.ds(ks, BAND), :]
                    s = lax.dot_general(
                        qh, kh, (((1,), (1,)), ((), ())),
                        preferred_element_type=F32) * SCALE
                    s = jnp.where(mask, s, -1e9)
                    mx = jnp.max(s, axis=1, keepdims=True)
                    w = jnp.exp(s - mx)
                    w = w / jnp.sum(w, axis=1, keepdims=True)
                    ch = lax.dot_general(
                        w.astype(BF16), vb_ref[h, pl.ds(ks, BAND), :],
                        (((1,), (0,)), ((), ())),
                        preferred_element_type=F32)
                    ctx_ref[:, h * DH:(h + 1) * DH] = ch.astype(BF16)
                oc = jnp.dot(ctx_ref[:], wo_b, preferred_element_type=F32)
                out_ref[0, pl.ds(c * CHUNK, CHUNK), :] = oc
                outb_ref[pl.ds(c * CHUNK, CHUNK), :] = oc.astype(BF16)
                for ti, tgt in enumerate((1, 3)):
                    rd = chunk_desc(c, ti, tgt)
                    rd.start()
                    sends.append(rd)
            for rd in sends:
                rd.wait_send()

        @pl.when(me == 1)
        def _dev1():
            for h in range(HQ):
                hk_ref[h] = k_ref[0, pl.ds(0, HALO), h, :].astype(BF16)
                hv_ref[h] = v_ref[0, pl.ds(0, HALO), h, :].astype(BF16)
            pend = []
            for i, (src, band) in enumerate(
                    ((hk_ref, kb_ref), (hv_ref, vb_ref))):
                rd = halo_desc(band, src, i)
                rd.start()
                pend.append(rd)
            for c in range(NCH):
                chunk_desc(c, 0, 0).wait_recv()
                fr = fwd_desc(c, 0, 0, fwda, 2)
                fr.start()
                pend.append(fr)
                out_ref[0, pl.ds(c * CHUNK, CHUNK), :] = (
                    outb_ref[pl.ds(c * CHUNK, CHUNK), :].astype(F32))
            for rd in pend:
                rd.wait_send()

        @pl.when(me == 3)
        def _dev3():
            pend = []
            for c in range(NCH):
                chunk_desc(c, 0, 0).wait_recv()
                fr = fwd_desc(c, HALF, 1, fwdb, 2)
                fr.start()
                pend.append(fr)
                out_ref[0, pl.ds(c * CHUNK, CHUNK), :] = (
                    outb_ref[pl.ds(c * CHUNK, CHUNK), :].astype(F32))
            for rd in pend:
                rd.wait_send()

        @pl.when(me == 2)
        def _dev2():
            for c in range(NCH):
                fwd_desc(c, 0, 0, fwda, 1).wait_recv()
                fwd_desc(c, HALF, 1, fwdb, 3).wait_recv()
                out_ref[0, pl.ds(c * CHUNK, CHUNK), :] = (
                    outb_ref[pl.ds(c * CHUNK, CHUNK), :].astype(F32))

    return pl.pallas_call(
        body,
        out_shape=jax.ShapeDtypeStruct((1, SQ, DM), F32),
        in_specs=[
            pl.BlockSpec(memory_space=pltpu.ANY),
            pl.BlockSpec(memory_space=pltpu.VMEM),
            pl.BlockSpec(memory_space=pltpu.VMEM),
            pl.BlockSpec(memory_space=pltpu.VMEM),
            pl.BlockSpec(memory_space=pltpu.VMEM),
        ],
        out_specs=pl.BlockSpec(memory_space=pltpu.VMEM),
        scratch_shapes=[
            pltpu.VMEM((HQ, KB, DH), BF16),
            pltpu.VMEM((HQ, KB, DH), BF16),
            pltpu.VMEM((SQ, DM), BF16),
            pltpu.VMEM((CHUNK, DM), F32),
            pltpu.VMEM((CHUNK, DM), BF16),
            pltpu.VMEM((HQ, HALO, DH), BF16),
            pltpu.VMEM((HQ, HALO, DH), BF16),
            pltpu.SemaphoreType.DMA((2,)),
            pltpu.SemaphoreType.DMA((2,)),
            pltpu.SemaphoreType.DMA((NCH, 2)),
            pltpu.SemaphoreType.DMA((NCH,)),
            pltpu.SemaphoreType.DMA((NCH,)),
            pltpu.SemaphoreType.DMA((NCH,)),
            pltpu.SemaphoreType.DMA,
        ],
        compiler_params=pltpu.CompilerParams(collective_id=7),
    )(x, Wq, K_ext, V_ext, Wo)


# baseline (device time: 107943 ns/iter reference)
import jax
import jax.numpy as jnp
from jax import lax
from jax.experimental import pallas as pl
from jax.experimental.pallas import tpu as pltpu

N_DEV = 4
SQ, DM, HQ, DH = 2048, 1024, 8, 128
CHUNK = 512
NCH = SQ // CHUNK
BAND = 768
WIN = 128
HALO = 128
KB = 2048 + HALO
HALF = CHUNK // 2
SCALE = 0.08838834764831843
F32, BF16 = jnp.float32, jnp.bfloat16
MESH = pl.DeviceIdType.MESH


def kernel(x, Wq, K_ext, V_ext, Wo):
    def body(x_hbm, wq_ref, k_hbm, v_hbm, wo_ref, out_ref,
             kb_ref, vb_ref, outb_ref, xst_ref, ctx_ref, hk_ref, hv_ref,
             kst_ref, vst_ref,
             halo_recv, halo_send, csend, crecv, fwda, fwdb, copy_sem):
        me = lax.axis_index("i")

        bsem = pltpu.get_barrier_semaphore()
        for off in (1, 2, 3):
            pl.semaphore_signal(
                bsem, inc=1, device_id=((me + off) % N_DEV,),
                device_id_type=MESH)
        pl.semaphore_wait(bsem, N_DEV - 1)

        def halo_desc(band_ref, src_ref, i):
            return pltpu.make_async_remote_copy(
                src_ref=src_ref,
                dst_ref=band_ref.at[:, pl.ds(2048, HALO), :],
                send_sem=halo_send.at[i], recv_sem=halo_recv.at[i],
                device_id=(0,), device_id_type=MESH)

        def chunk_desc(c, ti, tgt):
            sl = outb_ref.at[pl.ds(c * CHUNK, CHUNK), :]
            return pltpu.make_async_remote_copy(
                src_ref=sl, dst_ref=sl,
                send_sem=csend.at[c, ti], recv_sem=crecv.at[c],
                device_id=(tgt,), device_id_type=MESH)

        def fwd_desc(c, off, ti, sem, tgt):
            sl = outb_ref.at[pl.ds(c * CHUNK + off, HALF), :]
            return pltpu.make_async_remote_copy(
                src_ref=sl, dst_ref=sl,
                send_sem=csend.at[c, ti], recv_sem=sem.at[c],
                device_id=(tgt,), device_id_type=MESH)

        @pl.when(me == 0)
        def _dev0():
            wq_b = wq_ref[:].astype(BF16)
            wo_b = wo_ref[:].astype(BF16)
            for j in range(NCH):
                for src, st, band in ((k_hbm, kst_ref, kb_ref),
                                      (v_hbm, vst_ref, vb_ref)):
                    cp = pltpu.make_async_copy(
                        src.at[0, pl.ds(j * CHUNK, CHUNK)], st, copy_sem)
                    cp.start()
                    cp.wait()
                    for h in range(HQ):
                        band[h, pl.ds(j * CHUNK, CHUNK), :] = (
                            st[:, h, :].astype(BF16))
            sends = []
            for c in range(NCH):
                cp = pltpu.make_async_copy(
                    x_hbm.at[0, pl.ds(c * CHUNK, CHUNK), :], xst_ref,
                    copy_sem)
                cp.start()
                cp.wait()
                q = jnp.dot(xst_ref[:].astype(BF16), wq_b,
                            preferred_element_type=F32)
                qb = q.astype(BF16)
                if c == NCH - 1:
                    halo_desc(kb_ref, hk_ref, 0).wait_recv()
                    halo_desc(vb_ref, hv_ref, 1).wait_recv()
                ks = max(0, c * CHUNK - WIN)
                qi = c * CHUNK + lax.broadcasted_iota(
                    jnp.int32, (CHUNK, BAND), 0)
                ki = ks + lax.broadcasted_iota(jnp.int32, (CHUNK, BAND), 1)
                mask = jnp.abs(qi - ki) <= WIN
                for h in range(HQ):
                    qh = qb[:, h * DH:(h + 1) * DH]
                    kh = kb_ref[h, pl.ds(ks, BAND), :]
                    s = lax.dot_general(
                        qh, kh, (((1,), (1,)), ((), ())),
                        preferred_element_type=F32) * SCALE
                    s = jnp.where(mask, s, -1e9)
                    mx = jnp.max(s, axis=1, keepdims=True)
                    w = jnp.exp(s - mx)
                    w = w / jnp.sum(w, axis=1, keepdims=True)
                    ch = lax.dot_general(
                        w.astype(BF16), vb_ref[h, pl.ds(ks, BAND), :],
                        (((1,), (0,)), ((), ())),
                        preferred_element_type=F32)
                    ctx_ref[:, h * DH:(h + 1) * DH] = ch.astype(BF16)
                oc = jnp.dot(ctx_ref[:], wo_b, preferred_element_type=F32)
                out_ref[0, pl.ds(c * CHUNK, CHUNK), :] = oc
                outb_ref[pl.ds(c * CHUNK, CHUNK), :] = oc.astype(BF16)
                for ti, tgt in enumerate((1, 3)):
                    rd = chunk_desc(c, ti, tgt)
                    rd.start()
                    sends.append(rd)
            for rd in sends:
                rd.wait_send()

        @pl.when(me == 1)
        def _dev1():
            for src, st, halo in ((k_hbm, kst_ref, hk_ref),
                                  (v_hbm, vst_ref, hv_ref)):
                cp = pltpu.make_async_copy(
                    src.at[0, pl.ds(0, HALO)], st.at[pl.ds(0, HALO)],
                    copy_sem)
                cp.start()
                cp.wait()
                for h in range(HQ):
                    halo[h] = st[pl.ds(0, HALO), h, :].astype(BF16)
            pend = []
            for i, (src, band) in enumerate(
                    ((hk_ref, kb_ref), (hv_ref, vb_ref))):
                rd = halo_desc(band, src, i)
                rd.start()
                pend.append(rd)
            for c in range(NCH):
                chunk_desc(c, 0, 0).wait_recv()
                fr = fwd_desc(c, 0, 0, fwda, 2)
                fr.start()
                pend.append(fr)
                out_ref[0, pl.ds(c * CHUNK, CHUNK), :] = (
                    outb_ref[pl.ds(c * CHUNK, CHUNK), :].astype(F32))
            for rd in pend:
                rd.wait_send()

        @pl.when(me == 3)
        def _dev3():
            pend = []
            for c in range(NCH):
                chunk_desc(c, 0, 0).wait_recv()
                fr = fwd_desc(c, HALF, 1, fwdb, 2)
                fr.start()
                pend.append(fr)
                out_ref[0, pl.ds(c * CHUNK, CHUNK), :] = (
                    outb_ref[pl.ds(c * CHUNK, CHUNK), :].astype(F32))
            for rd in pend:
                rd.wait_send()

        @pl.when(me == 2)
        def _dev2():
            for c in range(NCH):
                fwd_desc(c, 0, 0, fwda, 1).wait_recv()
                fwd_desc(c, HALF, 1, fwdb, 3).wait_recv()
                out_ref[0, pl.ds(c * CHUNK, CHUNK), :] = (
                    outb_ref[pl.ds(c * CHUNK, CHUNK), :].astype(F32))

    return pl.pallas_call(
        body,
        out_shape=jax.ShapeDtypeStruct((1, SQ, DM), F32),
        in_specs=[
            pl.BlockSpec(memory_space=pl.ANY),
            pl.BlockSpec(memory_space=pltpu.VMEM),
            pl.BlockSpec(memory_space=pl.ANY),
            pl.BlockSpec(memory_space=pl.ANY),
            pl.BlockSpec(memory_space=pltpu.VMEM),
        ],
        out_specs=pl.BlockSpec(memory_space=pltpu.VMEM),
        scratch_shapes=[
            pltpu.VMEM((HQ, KB, DH), BF16),
            pltpu.VMEM((HQ, KB, DH), BF16),
            pltpu.VMEM((SQ, DM), BF16),
            pltpu.VMEM((CHUNK, DM), F32),
            pltpu.VMEM((CHUNK, DM), BF16),
            pltpu.VMEM((HQ, HALO, DH), BF16),
            pltpu.VMEM((HQ, HALO, DH), BF16),
            pltpu.VMEM((CHUNK, HQ, DH), F32),
            pltpu.VMEM((CHUNK, HQ, DH), F32),
            pltpu.SemaphoreType.DMA((2,)),
            pltpu.SemaphoreType.DMA((2,)),
            pltpu.SemaphoreType.DMA((NCH, 2)),
            pltpu.SemaphoreType.DMA((NCH,)),
            pltpu.SemaphoreType.DMA((NCH,)),
            pltpu.SemaphoreType.DMA((NCH,)),
            pltpu.SemaphoreType.DMA,
        ],
        compiler_params=pltpu.CompilerParams(
            collective_id=7, vmem_limit_bytes=44 * 1024 * 1024),
    )(x, Wq, K_ext, V_ext, Wo)


# device time: 88339 ns/iter; 1.2219x vs baseline; 1.2219x over previous
import jax
import jax.numpy as jnp
from jax import lax
from jax.experimental import pallas as pl
from jax.experimental.pallas import tpu as pltpu

N_DEV = 4
SQ, DM, HQ, DH = 2048, 1024, 8, 128
CHUNK = 512
NCH = SQ // CHUNK
BAND = 768
WIN = 128
HALO = 128
KB = 2048 + HALO
HALF = CHUNK // 2
SCALE = 0.08838834764831843
NEG = -30.0
F32, BF16 = jnp.float32, jnp.bfloat16
MESH = pl.DeviceIdType.MESH


def kernel(x, Wq, K_ext, V_ext, Wo):
    def body(x_hbm, wq_ref, k_hbm, v_hbm, wo_ref, out_ref,
             kb_ref, vb_ref, outb_ref, xst_ref, ctx_ref, hk_ref, hv_ref,
             kvst_ref,
             halo_recv, halo_send, csend, crecv, fwda, fwdb,
             copy_sems, xcopy_sems):
        me = lax.axis_index("i")

        bsem = pltpu.get_barrier_semaphore()
        for off in (1, 2, 3):
            pl.semaphore_signal(
                bsem, inc=1, device_id=((me + off) % N_DEV,),
                device_id_type=MESH)
        pl.semaphore_wait(bsem, N_DEV - 1)

        def halo_desc(band_ref, src_ref, i):
            return pltpu.make_async_remote_copy(
                src_ref=src_ref,
                dst_ref=band_ref.at[:, pl.ds(2048, HALO), :],
                send_sem=halo_send.at[i], recv_sem=halo_recv.at[i],
                device_id=(0,), device_id_type=MESH)

        def half_desc(c, half, ti, tgt):
            sl = outb_ref.at[pl.ds(c * CHUNK + half * HALF, HALF), :]
            return pltpu.make_async_remote_copy(
                src_ref=sl, dst_ref=sl,
                send_sem=csend.at[c, 2 * half + ti],
                recv_sem=crecv.at[c, half],
                device_id=(tgt,), device_id_type=MESH)

        def fwd_desc(c, half, ti, sem, tgt):
            sl = outb_ref.at[pl.ds(c * CHUNK + half * HALF, HALF), :]
            return pltpu.make_async_remote_copy(
                src_ref=sl, dst_ref=sl,
                send_sem=csend.at[c, ti], recv_sem=sem.at[c],
                device_id=(tgt,), device_id_type=MESH)

        @pl.when(me == 0)
        def _dev0():
            wq_b = wq_ref[:].astype(BF16)
            wo_b = wo_ref[:].astype(BF16)
            steps = [(src, band, h)
                     for src, band in ((k_hbm, kb_ref), (v_hbm, vb_ref))
                     for h in range(HQ)]

            def start_fill(i):
                src, _, h = steps[i]
                cp = pltpu.make_async_copy(
                    src.at[0, :, h, :], kvst_ref.at[i % 2],
                    copy_sems.at[i % 2])
                cp.start()
                return cp

            pend_fill = [start_fill(0), start_fill(1)]
            xcp = pltpu.make_async_copy(
                x_hbm.at[0, pl.ds(0, CHUNK), :], xst_ref.at[0],
                xcopy_sems.at[0])
            xcp.start()
            for i, (_, band, h) in enumerate(steps):
                pend_fill[i % 2].wait()
                band[h, pl.ds(0, 2048), :] = kvst_ref[i % 2].astype(BF16)
                if i + 2 < len(steps):
                    pend_fill[i % 2] = start_fill(i + 2)

            sends = []
            for c in range(NCH):
                pltpu.make_async_copy(
                    x_hbm.at[0, pl.ds(c * CHUNK, CHUNK), :],
                    xst_ref.at[c % 2], xcopy_sems.at[c % 2]).wait()
                if c + 1 < NCH:
                    pltpu.make_async_copy(
                        x_hbm.at[0, pl.ds((c + 1) * CHUNK, CHUNK), :],
                        xst_ref.at[(c + 1) % 2],
                        xcopy_sems.at[(c + 1) % 2]).start()
                q = jnp.dot(xst_ref[c % 2].astype(BF16), wq_b,
                            preferred_element_type=F32)
                qb = (q * SCALE).astype(BF16)
                if c == NCH - 1:
                    halo_desc(kb_ref, hk_ref, 0).wait_recv()
                    halo_desc(vb_ref, hv_ref, 1).wait_recv()
                ks = max(0, c * CHUNK - WIN)
                qi = c * CHUNK + lax.broadcasted_iota(
                    jnp.int32, (CHUNK, BAND), 0)
                ki = ks + lax.broadcasted_iota(jnp.int32, (CHUNK, BAND), 1)
                mask = jnp.abs(qi - ki) <= WIN
                for h in range(HQ):
                    qh = qb[:, h * DH:(h + 1) * DH]
                    kh = kb_ref[h, pl.ds(ks, BAND), :]
                    s = lax.dot_general(
                        qh, kh, (((1,), (1,)), ((), ())),
                        preferred_element_type=F32)
                    w = jnp.exp(jnp.where(mask, s, NEG))
                    w = w / jnp.sum(w, axis=1, keepdims=True)
                    ch = lax.dot_general(
                        w.astype(BF16), vb_ref[h, pl.ds(ks, BAND), :],
                        (((1,), (0,)), ((), ())),
                        preferred_element_type=F32)
                    ctx_ref[:, h * DH:(h + 1) * DH] = ch.astype(BF16)
                oc = jnp.dot(ctx_ref[:], wo_b, preferred_element_type=F32)
                outb_ref[pl.ds(c * CHUNK, CHUNK), :] = oc.astype(BF16)
                for half in range(2):
                    for ti, tgt in enumerate((1, 3)):
                        rd = half_desc(c, half, ti, tgt)
                        rd.start()
                        sends.append(rd)
                out_ref[0, pl.ds(c * CHUNK, CHUNK), :] = oc
            for rd in sends:
                rd.wait_send()

        @pl.when(me == 1)
        def _dev1():
            for src, halo in ((k_hbm, hk_ref), (v_hbm, hv_ref)):
                for h in range(HQ):
                    cp = pltpu.make_async_copy(
                        src.at[0, pl.ds(0, HALO), h, :],
                        kvst_ref.at[0, pl.ds(0, HALO)], copy_sems.at[0])
                    cp.start()
                    cp.wait()
                    halo[h] = kvst_ref[0, pl.ds(0, HALO), :].astype(BF16)
            pend = []
            for i, (src, band) in enumerate(
                    ((hk_ref, kb_ref), (hv_ref, vb_ref))):
                rd = halo_desc(band, src, i)
                rd.start()
                pend.append(rd)
            for c in range(NCH):
                half_desc(c, 0, 0, 0).wait_recv()
                fr = fwd_desc(c, 0, 0, fwda, 2)
                fr.start()
                pend.append(fr)
                out_ref[0, pl.ds(c * CHUNK, HALF), :] = (
                    outb_ref[pl.ds(c * CHUNK, HALF), :].astype(F32))
                half_desc(c, 1, 0, 0).wait_recv()
                out_ref[0, pl.ds(c * CHUNK + HALF, HALF), :] = (
                    outb_ref[pl.ds(c * CHUNK + HALF, HALF), :].astype(F32))
            for rd in pend:
                rd.wait_send()

        @pl.when(me == 3)
        def _dev3():
            pend = []
            for c in range(NCH):
                half_desc(c, 1, 0, 0).wait_recv()
                fr = fwd_desc(c, 1, 1, fwdb, 2)
                fr.start()
                pend.append(fr)
                out_ref[0, pl.ds(c * CHUNK + HALF, HALF), :] = (
                    outb_ref[pl.ds(c * CHUNK + HALF, HALF), :].astype(F32))
                half_desc(c, 0, 0, 0).wait_recv()
                out_ref[0, pl.ds(c * CHUNK, HALF), :] = (
                    outb_ref[pl.ds(c * CHUNK, HALF), :].astype(F32))
            for rd in pend:
                rd.wait_send()

        @pl.when(me == 2)
        def _dev2():
            for c in range(NCH):
                fwd_desc(c, 0, 0, fwda, 1).wait_recv()
                out_ref[0, pl.ds(c * CHUNK, HALF), :] = (
                    outb_ref[pl.ds(c * CHUNK, HALF), :].astype(F32))
                fwd_desc(c, 1, 1, fwdb, 3).wait_recv()
                out_ref[0, pl.ds(c * CHUNK + HALF, HALF), :] = (
                    outb_ref[pl.ds(c * CHUNK + HALF, HALF), :].astype(F32))

    return pl.pallas_call(
        body,
        out_shape=jax.ShapeDtypeStruct((1, SQ, DM), F32),
        in_specs=[
            pl.BlockSpec(memory_space=pl.ANY),
            pl.BlockSpec(memory_space=pltpu.VMEM),
            pl.BlockSpec(memory_space=pl.ANY),
            pl.BlockSpec(memory_space=pl.ANY),
            pl.BlockSpec(memory_space=pltpu.VMEM),
        ],
        out_specs=pl.BlockSpec(memory_space=pltpu.VMEM),
        scratch_shapes=[
            pltpu.VMEM((HQ, KB, DH), BF16),
            pltpu.VMEM((HQ, KB, DH), BF16),
            pltpu.VMEM((SQ, DM), BF16),
            pltpu.VMEM((2, CHUNK, DM), F32),
            pltpu.VMEM((CHUNK, DM), BF16),
            pltpu.VMEM((HQ, HALO, DH), BF16),
            pltpu.VMEM((HQ, HALO, DH), BF16),
            pltpu.VMEM((2, 2048, DH), F32),
            pltpu.SemaphoreType.DMA((2,)),
            pltpu.SemaphoreType.DMA((2,)),
            pltpu.SemaphoreType.DMA((NCH, 4)),
            pltpu.SemaphoreType.DMA((NCH, 2)),
            pltpu.SemaphoreType.DMA((NCH,)),
            pltpu.SemaphoreType.DMA((NCH,)),
            pltpu.SemaphoreType.DMA((2,)),
            pltpu.SemaphoreType.DMA((2,)),
        ],
        compiler_params=pltpu.CompilerParams(
            collective_id=7, vmem_limit_bytes=46 * 1024 * 1024),
    )(x, Wq, K_ext, V_ext, Wo)


# device time: 84984 ns/iter; 1.2702x vs baseline; 1.0395x over previous
import jax
import jax.numpy as jnp
from jax import lax
from jax.experimental import pallas as pl
from jax.experimental.pallas import tpu as pltpu

N_DEV = 4
SQ, DM, HQ, DH = 2048, 1024, 8, 128
CHUNK = 512
NCH = SQ // CHUNK
BAND = 768
WIN = 128
HALO = 128
KB = 2048 + HALO
HALF = CHUNK // 2
SCALE = 0.08838834764831843
NEG = -30.0
F32, BF16 = jnp.float32, jnp.bfloat16
MESH = pl.DeviceIdType.MESH


def kernel(x, Wq, K_ext, V_ext, Wo):
    def body(x_hbm, wq_ref, k_hbm, v_hbm, wo_ref, out_ref,
             kb_ref, vb_ref, outb_ref, xst_ref, ctx_ref, hk_ref, hv_ref,
             kvst_ref,
             halo_recv, halo_send, csend, crecv, fwda, fwdb,
             copy_sems, xcopy_sems):
        me = lax.axis_index("i")

        bsem = pltpu.get_barrier_semaphore()
        for off in (1, 2, 3):
            pl.semaphore_signal(
                bsem, inc=1, device_id=((me + off) % N_DEV,),
                device_id_type=MESH)
        pl.semaphore_wait(bsem, N_DEV - 1)

        def halo_desc(band_ref, src_ref, i):
            return pltpu.make_async_remote_copy(
                src_ref=src_ref,
                dst_ref=band_ref.at[pl.ds(2048, HALO), :],
                send_sem=halo_send.at[i], recv_sem=halo_recv.at[i],
                device_id=(0,), device_id_type=MESH)

        def half_desc(c, half, ti, tgt):
            sl = outb_ref.at[pl.ds(c * CHUNK + half * HALF, HALF), :]
            return pltpu.make_async_remote_copy(
                src_ref=sl, dst_ref=sl,
                send_sem=csend.at[c, 2 * half + ti],
                recv_sem=crecv.at[c, half],
                device_id=(tgt,), device_id_type=MESH)

        def fwd_desc(c, half, ti, sem, tgt):
            sl = outb_ref.at[pl.ds(c * CHUNK + half * HALF, HALF), :]
            return pltpu.make_async_remote_copy(
                src_ref=sl, dst_ref=sl,
                send_sem=csend.at[c, ti], recv_sem=sem.at[c],
                device_id=(tgt,), device_id_type=MESH)

        @pl.when(me == 0)
        def _dev0():
            wq_b = wq_ref[:].astype(BF16)
            wo_b = wo_ref[:].astype(BF16)
            steps = [(src, band, j)
                     for j in range(NCH)
                     for src, band in ((k_hbm, kb_ref), (v_hbm, vb_ref))]

            def start_fill(i):
                src, _, j = steps[i]
                cp = pltpu.make_async_copy(
                    src.at[0, pl.ds(j * CHUNK, CHUNK)],
                    kvst_ref.at[i % 2], copy_sems.at[i % 2])
                cp.start()
                return cp

            pend_fill = [start_fill(0), start_fill(1)]
            xcp = pltpu.make_async_copy(
                x_hbm.at[0, pl.ds(0, CHUNK), :], xst_ref.at[0],
                xcopy_sems.at[0])
            xcp.start()
            for i, (_, band, j) in enumerate(steps):
                pend_fill[i % 2].wait()
                band[pl.ds(j * CHUNK, CHUNK), :] = (
                    kvst_ref[i % 2].reshape(CHUNK, DM).astype(BF16))
                if i + 2 < len(steps):
                    pend_fill[i % 2] = start_fill(i + 2)

            sends = []
            for c in range(NCH):
                pltpu.make_async_copy(
                    x_hbm.at[0, pl.ds(c * CHUNK, CHUNK), :],
                    xst_ref.at[c % 2], xcopy_sems.at[c % 2]).wait()
                if c + 1 < NCH:
                    pltpu.make_async_copy(
                        x_hbm.at[0, pl.ds((c + 1) * CHUNK, CHUNK), :],
                        xst_ref.at[(c + 1) % 2],
                        xcopy_sems.at[(c + 1) % 2]).start()
                q = jnp.dot(xst_ref[c % 2].astype(BF16), wq_b,
                            preferred_element_type=F32)
                qb = (q * SCALE).astype(BF16)
                if c == NCH - 1:
                    halo_desc(kb_ref, hk_ref, 0).wait_recv()
                    halo_desc(vb_ref, hv_ref, 1).wait_recv()
                ks = max(0, c * CHUNK - WIN)
                qi = c * CHUNK + lax.broadcasted_iota(
                    jnp.int32, (CHUNK, BAND), 0)
                ki = ks + lax.broadcasted_iota(jnp.int32, (CHUNK, BAND), 1)
                mask = jnp.abs(qi - ki) <= WIN
                for h in range(HQ):
                    qh = qb[:, h * DH:(h + 1) * DH]
                    kh = kb_ref[pl.ds(ks, BAND), h * DH:(h + 1) * DH]
                    s = lax.dot_general(
                        qh, kh, (((1,), (1,)), ((), ())),
                        preferred_element_type=F32)
                    w = jnp.exp(jnp.where(mask, s, NEG))
                    inv = 1.0 / jnp.sum(w, axis=1, keepdims=True)
                    ch = lax.dot_general(
                        w.astype(BF16),
                        vb_ref[pl.ds(ks, BAND), h * DH:(h + 1) * DH],
                        (((1,), (0,)), ((), ())),
                        preferred_element_type=F32)
                    ctx_ref[:, h * DH:(h + 1) * DH] = (ch * inv).astype(BF16)
                oc = jnp.dot(ctx_ref[:], wo_b, preferred_element_type=F32)
                outb_ref[pl.ds(c * CHUNK, CHUNK), :] = oc.astype(BF16)
                for half in range(2):
                    for ti, tgt in enumerate((1, 3)):
                        rd = half_desc(c, half, ti, tgt)
                        rd.start()
                        sends.append(rd)
                out_ref[0, pl.ds(c * CHUNK, CHUNK), :] = oc
            for rd in sends:
                rd.wait_send()

        @pl.when(me == 1)
        def _dev1():
            for src, halo in ((k_hbm, hk_ref), (v_hbm, hv_ref)):
                cp = pltpu.make_async_copy(
                    src.at[0, pl.ds(0, HALO)],
                    kvst_ref.at[0, pl.ds(0, HALO)], copy_sems.at[0])
                cp.start()
                cp.wait()
                halo[:] = (kvst_ref[0, pl.ds(0, HALO), :, :]
                           .reshape(HALO, DM).astype(BF16))
            pend = []
            for i, (src, band) in enumerate(
                    ((hk_ref, kb_ref), (hv_ref, vb_ref))):
                rd = halo_desc(band, src, i)
                rd.start()
                pend.append(rd)
            for c in range(NCH):
                half_desc(c, 0, 0, 0).wait_recv()
                fr = fwd_desc(c, 0, 0, fwda, 2)
                fr.start()
                pend.append(fr)
                out_ref[0, pl.ds(c * CHUNK, HALF), :] = (
                    outb_ref[pl.ds(c * CHUNK, HALF), :].astype(F32))
                half_desc(c, 1, 0, 0).wait_recv()
                out_ref[0, pl.ds(c * CHUNK + HALF, HALF), :] = (
                    outb_ref[pl.ds(c * CHUNK + HALF, HALF), :].astype(F32))
            for rd in pend:
                rd.wait_send()

        @pl.when(me == 3)
        def _dev3():
            pend = []
            for c in range(NCH):
                half_desc(c, 1, 0, 0).wait_recv()
                fr = fwd_desc(c, 1, 1, fwdb, 2)
                fr.start()
                pend.append(fr)
                out_ref[0, pl.ds(c * CHUNK + HALF, HALF), :] = (
                    outb_ref[pl.ds(c * CHUNK + HALF, HALF), :].astype(F32))
                half_desc(c, 0, 0, 0).wait_recv()
                out_ref[0, pl.ds(c * CHUNK, HALF), :] = (
                    outb_ref[pl.ds(c * CHUNK, HALF), :].astype(F32))
            for rd in pend:
                rd.wait_send()

        @pl.when(me == 2)
        def _dev2():
            for c in range(NCH):
                fwd_desc(c, 0, 0, fwda, 1).wait_recv()
                out_ref[0, pl.ds(c * CHUNK, HALF), :] = (
                    outb_ref[pl.ds(c * CHUNK, HALF), :].astype(F32))
                fwd_desc(c, 1, 1, fwdb, 3).wait_recv()
                out_ref[0, pl.ds(c * CHUNK + HALF, HALF), :] = (
                    outb_ref[pl.ds(c * CHUNK + HALF, HALF), :].astype(F32))

    return pl.pallas_call(
        body,
        out_shape=jax.ShapeDtypeStruct((1, SQ, DM), F32),
        in_specs=[
            pl.BlockSpec(memory_space=pl.ANY),
            pl.BlockSpec(memory_space=pltpu.VMEM),
            pl.BlockSpec(memory_space=pl.ANY),
            pl.BlockSpec(memory_space=pl.ANY),
            pl.BlockSpec(memory_space=pltpu.VMEM),
        ],
        out_specs=pl.BlockSpec(memory_space=pltpu.VMEM),
        scratch_shapes=[
            pltpu.VMEM((KB, DM), BF16),
            pltpu.VMEM((KB, DM), BF16),
            pltpu.VMEM((SQ, DM), BF16),
            pltpu.VMEM((2, CHUNK, DM), F32),
            pltpu.VMEM((CHUNK, DM), BF16),
            pltpu.VMEM((HALO, DM), BF16),
            pltpu.VMEM((HALO, DM), BF16),
            pltpu.VMEM((2, CHUNK, HQ, DH), F32),
            pltpu.SemaphoreType.DMA((2,)),
            pltpu.SemaphoreType.DMA((2,)),
            pltpu.SemaphoreType.DMA((NCH, 4)),
            pltpu.SemaphoreType.DMA((NCH, 2)),
            pltpu.SemaphoreType.DMA((NCH,)),
            pltpu.SemaphoreType.DMA((NCH,)),
            pltpu.SemaphoreType.DMA((2,)),
            pltpu.SemaphoreType.DMA((2,)),
        ],
        compiler_params=pltpu.CompilerParams(
            collective_id=7, vmem_limit_bytes=47 * 1024 * 1024),
    )(x, Wq, K_ext, V_ext, Wo)
